# baseline (speedup 1.0000x reference)
import functools

import jax
import jax.numpy as jnp
from jax import lax
from jax.experimental import pallas as pl
from jax.experimental.pallas import tpu as pltpu
from jax.experimental.pallas import tpu_sc as plsc

NC = 2
NS = 16
NW = NC * NS

_ZROWS = 25


@functools.lru_cache(maxsize=None)
def _make_segsum(n, e, d, chunk, nbuf):
    assert e % (NW * chunk) == 0
    e_per = e // NW
    nch = e_per // chunk
    assert nch % nbuf == 0
    nit = nch // nbuf
    rps = n // NS
    assert rps % _ZROWS == 0
    mesh = plsc.VectorSubcoreMesh(
        core_axis_name="c", subcore_axis_name="s", num_cores=NC, num_subcores=NS
    )

    @functools.partial(
        pl.kernel,
        mesh=mesh,
        out_type=jax.ShapeDtypeStruct((NC, n, d), jnp.float32),
        scratch_types=[
            pltpu.VMEM((nch, chunk), jnp.int32),
            pltpu.VMEM((nch, chunk), jnp.int32),
            [pltpu.VMEM((chunk, d), jnp.float32) for _ in range(nbuf)],
            pltpu.VMEM((_ZROWS, d), jnp.float32),
            pltpu.VMEM_SHARED((n, d), jnp.float32),
            [pltpu.SemaphoreType.DMA for _ in range(nbuf)],
            [pltpu.SemaphoreType.DMA for _ in range(nbuf)],
        ],
        compiler_params=pltpu.CompilerParams(
            use_tc_tiling_on_sc=False,
            disable_bounds_checks=True,
            disable_semaphore_checks=True,
            skip_device_barrier=True,
        ),
    )
    def segsum(x_hbm, ei_hbm, out_hbm,
               src_v, dst_v, rows, zbuf, acc, sem_g, sem_s):
        c = lax.axis_index("c")
        s = lax.axis_index("s")
        wid = s * NC + c
        row0 = s * rps

        pltpu.async_copy(ei_hbm.at[0, pl.ds(wid * nch, nch)], src_v, sem_g[0])
        pltpu.async_copy(ei_hbm.at[1, pl.ds(wid * nch, nch)], dst_v, sem_g[1])

        def zrow(r, carry):
            def zcol(q, carry2):
                zbuf[r, pl.ds(q * 16, 16)] = jnp.zeros((16,), jnp.float32)
                return carry2
            return lax.fori_loop(0, d // 16, zcol, carry)

        lax.fori_loop(0, _ZROWS, zrow, 0)

        nz = rps // _ZROWS

        def zcopy(i, carry):
            pltpu.async_copy(zbuf, acc.at[pl.ds(row0 + i * _ZROWS, _ZROWS)],
                             sem_s[0])
            return carry

        lax.fori_loop(0, nz, zcopy, 0)

        def zdrain(i, carry):
            pltpu.make_async_copy(zbuf, acc.at[pl.ds(row0, _ZROWS)],
                                  sem_s[0]).wait()
            return carry

        pltpu.make_async_copy(ei_hbm.at[0, pl.ds(0, nch)], src_v,
                              sem_g[0]).wait()
        pltpu.make_async_copy(ei_hbm.at[0, pl.ds(0, nch)], dst_v,
                              sem_g[1]).wait()
        for b in range(nbuf):
            pltpu.async_copy(x_hbm.at[src_v.at[b]], rows[b], sem_g[b])

        lax.fori_loop(0, nz, zdrain, 0)
        plsc.subcore_barrier()

        def body(k, carry):
            j0 = k * nbuf
            for b in range(nbuf):
                pltpu.make_async_copy(
                    x_hbm.at[pl.ds(0, chunk)], rows[b], sem_g[b]).wait()
                pltpu.async_copy(rows[b], acc.at[dst_v.at[j0 + b]],
                                 sem_s[b], add=True)
            for b in range(nbuf):
                @pl.when(k < nit - 1)
                def _():
                    pltpu.make_async_copy(
                        rows[b], acc.at[pl.ds(0, chunk)], sem_s[b]).wait()
                    pltpu.async_copy(x_hbm.at[src_v.at[j0 + nbuf + b]],
                                     rows[b], sem_g[b])
            return carry

        lax.fori_loop(0, nit, body, 0)
        for b in range(nbuf):
            pltpu.make_async_copy(
                rows[b], acc.at[pl.ds(0, chunk)], sem_s[b]).wait()
        plsc.subcore_barrier()
        pltpu.sync_copy(acc.at[pl.ds(row0, rps)],
                        out_hbm.at[c, pl.ds(row0, rps)])

    return segsum


def _mid_body(p_ref, w0_ref, b0_ref, w1_ref, o_ref):
    ssum = p_ref[0] + p_ref[1]
    h = jnp.dot(ssum, w0_ref[...], preferred_element_type=jnp.float32)
    h = jnp.maximum(h + b0_ref[...], 0.0)
    o_ref[...] = jnp.dot(h, w1_ref[...], preferred_element_type=jnp.float32)


def _final_body(p_ref, b1_ref, o_ref):
    ssum = p_ref[0] + p_ref[1] + b1_ref[...]
    m = jnp.max(ssum, axis=1, keepdims=True)
    shifted = ssum - m
    o_ref[...] = shifted - jnp.log(jnp.sum(jnp.exp(shifted), axis=1, keepdims=True))


def kernel(x, edge_index, W0, b0, W1, b1):
    n, d = x.shape
    h_dim = W0.shape[1]
    o_dim = W1.shape[1]
    e = edge_index.shape[1]

    ei128 = edge_index.reshape(2, e // 40, 40)
    p1 = _make_segsum(n, e, d, 40, 5)(x, ei128)

    blk = 2000
    hw = pl.pallas_call(
        _mid_body,
        grid=(n // blk,),
        in_specs=[
            pl.BlockSpec((NC, blk, d), lambda i: (0, i, 0)),
            pl.BlockSpec((d, h_dim), lambda i: (0, 0)),
            pl.BlockSpec((1, h_dim), lambda i: (0, 0)),
            pl.BlockSpec((h_dim, o_dim), lambda i: (0, 0)),
        ],
        out_specs=pl.BlockSpec((blk, o_dim), lambda i: (i, 0)),
        out_shape=jax.ShapeDtypeStruct((n, o_dim), jnp.float32),
    )(p1, W0, b0.reshape(1, h_dim), W1)

    ei64 = edge_index.reshape(2, e // 80, 80)
    p2 = _make_segsum(n, e, o_dim, 80, 5)(hw, ei64)

    out = pl.pallas_call(
        _final_body,
        grid=(n // blk,),
        in_specs=[
            pl.BlockSpec((NC, blk, o_dim), lambda i: (0, i, 0)),
            pl.BlockSpec((1, o_dim), lambda i: (0, 0)),
        ],
        out_specs=pl.BlockSpec((blk, o_dim), lambda i: (i, 0)),
        out_shape=jax.ShapeDtypeStruct((n, o_dim), jnp.float32),
    )(p2, b1.reshape(1, o_dim))
    return out


# speedup vs baseline: 1.0127x; 1.0127x over previous
import functools

import jax
import jax.numpy as jnp
from jax import lax
from jax.experimental import pallas as pl
from jax.experimental.pallas import tpu as pltpu
from jax.experimental.pallas import tpu_sc as plsc

NC = 2
NS = 16
NW = NC * NS

_ZROWS = 25


@functools.lru_cache(maxsize=None)
def _make_segsum(n, e, d, chunk, nbuf):
    assert e % (NW * chunk) == 0
    e_per = e // NW
    nch = e_per // chunk
    assert nch % nbuf == 0
    nit = nch // nbuf
    rps = n // NS
    assert rps % _ZROWS == 0
    mesh = plsc.VectorSubcoreMesh(
        core_axis_name="c", subcore_axis_name="s", num_cores=NC, num_subcores=NS
    )

    @functools.partial(
        pl.kernel,
        mesh=mesh,
        out_type=jax.ShapeDtypeStruct((NC, n, d), jnp.float32),
        scratch_types=[
            pltpu.VMEM((nch, chunk), jnp.int32),
            pltpu.VMEM((nch, chunk), jnp.int32),
            [pltpu.VMEM((chunk, d), jnp.float32) for _ in range(nbuf)],
            pltpu.VMEM((_ZROWS, d), jnp.float32),
            pltpu.VMEM_SHARED((n, d), jnp.float32),
            [pltpu.SemaphoreType.DMA for _ in range(nbuf)],
            [pltpu.SemaphoreType.DMA for _ in range(nbuf)],
        ],
        compiler_params=pltpu.CompilerParams(
            use_tc_tiling_on_sc=False,
            disable_bounds_checks=True,
            disable_semaphore_checks=True,
        ),
    )
    def segsum(x_hbm, ei_hbm, out_hbm,
               src_v, dst_v, rows, zbuf, acc, sem_g, sem_s):
        c = lax.axis_index("c")
        s = lax.axis_index("s")
        wid = s * NC + c
        row0 = s * rps

        pltpu.async_copy(ei_hbm.at[0, pl.ds(wid * nch, nch)], src_v, sem_g[0])
        pltpu.async_copy(ei_hbm.at[1, pl.ds(wid * nch, nch)], dst_v, sem_g[1])

        def zrow(r, carry):
            def zcol(q, carry2):
                zbuf[r, pl.ds(q * 16, 16)] = jnp.zeros((16,), jnp.float32)
                return carry2
            return lax.fori_loop(0, d // 16, zcol, carry)

        lax.fori_loop(0, _ZROWS, zrow, 0)

        nz = rps // _ZROWS

        def zcopy(i, carry):
            pltpu.async_copy(zbuf, acc.at[pl.ds(row0 + i * _ZROWS, _ZROWS)],
                             sem_s[0])
            return carry

        lax.fori_loop(0, nz, zcopy, 0)

        def zdrain(i, carry):
            pltpu.make_async_copy(zbuf, acc.at[pl.ds(row0, _ZROWS)],
                                  sem_s[0]).wait()
            return carry

        pltpu.make_async_copy(ei_hbm.at[0, pl.ds(0, nch)], src_v,
                              sem_g[0]).wait()
        pltpu.make_async_copy(ei_hbm.at[0, pl.ds(0, nch)], dst_v,
                              sem_g[1]).wait()
        for b in range(nbuf):
            pltpu.async_copy(x_hbm.at[src_v.at[b]], rows[b], sem_g[b])

        lax.fori_loop(0, nz, zdrain, 0)
        plsc.subcore_barrier()

        def body(k, carry):
            j0 = k * nbuf
            for b in range(nbuf):
                pltpu.make_async_copy(
                    x_hbm.at[pl.ds(0, chunk)], rows[b], sem_g[b]).wait()
                pltpu.async_copy(rows[b], acc.at[dst_v.at[j0 + b]],
                                 sem_s[b], add=True)
            for b in range(nbuf):
                @pl.when(k < nit - 1)
                def _():
                    pltpu.make_async_copy(
                        rows[b], acc.at[pl.ds(0, chunk)], sem_s[b]).wait()
                    pltpu.async_copy(x_hbm.at[src_v.at[j0 + nbuf + b]],
                                     rows[b], sem_g[b])
            return carry

        lax.fori_loop(0, nit, body, 0)
        for b in range(nbuf):
            pltpu.make_async_copy(
                rows[b], acc.at[pl.ds(0, chunk)], sem_s[b]).wait()
        plsc.subcore_barrier()
        pltpu.sync_copy(acc.at[pl.ds(row0, rps)],
                        out_hbm.at[c, pl.ds(row0, rps)])

    return segsum


def _mid_body(p_ref, w0_ref, b0_ref, w1_ref, o_ref):
    ssum = p_ref[0] + p_ref[1]
    h = jnp.dot(ssum, w0_ref[...], preferred_element_type=jnp.float32)
    h = jnp.maximum(h + b0_ref[...], 0.0)
    o_ref[...] = jnp.dot(h, w1_ref[...], preferred_element_type=jnp.float32)


def _final_body(p_ref, b1_ref, o_ref):
    blk2 = p_ref.shape[1]
    ssum = (p_ref[0] + p_ref[1] + b1_ref[...]).reshape(blk2, 2, 64)
    m = jnp.max(ssum, axis=2, keepdims=True)
    shifted = ssum - m
    lse = jnp.log(jnp.sum(jnp.exp(shifted), axis=2, keepdims=True))
    o_ref[...] = (shifted - lse).reshape(blk2, 128)


def kernel(x, edge_index, W0, b0, W1, b1):
    n, d = x.shape
    h_dim = W0.shape[1]
    o_dim = W1.shape[1]
    e = edge_index.shape[1]

    ei128 = edge_index.reshape(2, e // 40, 40)
    p1 = _make_segsum(n, e, d, 40, 5)(x, ei128)

    blk = 2000
    hw = pl.pallas_call(
        _mid_body,
        grid=(n // blk,),
        in_specs=[
            pl.BlockSpec((NC, blk, d), lambda i: (0, i, 0)),
            pl.BlockSpec((d, h_dim), lambda i: (0, 0)),
            pl.BlockSpec((1, h_dim), lambda i: (0, 0)),
            pl.BlockSpec((h_dim, o_dim), lambda i: (0, 0)),
        ],
        out_specs=pl.BlockSpec((blk, o_dim), lambda i: (i, 0)),
        out_shape=jax.ShapeDtypeStruct((n, o_dim), jnp.float32),
    )(p1, W0, b0.reshape(1, h_dim), W1)

    ei64 = edge_index.reshape(2, e // 80, 80)
    p2 = _make_segsum(n, e, o_dim, 80, 5)(hw, ei64)

    n2 = n // 2
    blk_f = 1000
    p2r = p2.reshape(NC, n2, 2 * o_dim)
    b1r = jnp.tile(b1.reshape(1, o_dim), (1, 2))
    out = pl.pallas_call(
        _final_body,
        grid=(n2 // blk_f,),
        in_specs=[
            pl.BlockSpec((NC, blk_f, 2 * o_dim), lambda i: (0, i, 0)),
            pl.BlockSpec((1, 2 * o_dim), lambda i: (0, 0)),
        ],
        out_specs=pl.BlockSpec((blk_f, 2 * o_dim), lambda i: (i, 0)),
        out_shape=jax.ShapeDtypeStruct((n2, 2 * o_dim), jnp.float32),
    )(p2r, b1r)
    return out.reshape(n, o_dim)
